# Initial kernel scaffold; baseline (speedup 1.0000x reference)
import jax
import jax.numpy as jnp
from jax.experimental import pallas as pl


def kernel(inter_pre, bags_size):
    raise NotImplementedError("write your pallas kernel here")


# speedup vs baseline: 2.5765x; 2.5765x over previous
import functools

import jax
import jax.numpy as jnp
from jax import lax
from jax.experimental import pallas as pl
from jax.experimental.pallas import tpu as pltpu
from jax.experimental.pallas import tpu_sc as plsc

L = 16
CH = 128
NC = 2
NS = 16


def _make_seg_max(total, d, b):
    colh = d // 2
    ngrp = colh // L

    mesh = plsc.VectorSubcoreMesh(
        core_axis_name="c", subcore_axis_name="s",
        num_cores=NC, num_subcores=NS)

    @functools.partial(
        pl.kernel,
        out_type=jax.ShapeDtypeStruct((b, d), jnp.float32),
        mesh=mesh,
        scratch_types=[
            pltpu.VMEM((2, CH, colh), jnp.float32),
            pltpu.VMEM((L,), jnp.int32),
            pltpu.VMEM((1, colh), jnp.float32),
            pltpu.SemaphoreType.DMA,
        ],
        compiler_params=pltpu.CompilerParams(
            use_tc_tiling_on_sc=False, needs_layout_passes=False),
    )
    def seg_max(x_hbm, sizes_hbm, out_hbm, buf, sz_v, out_v, sem):
        wid = lax.axis_index("s") * NC + lax.axis_index("c")
        bag = wid // 2
        half = wid % 2
        col0 = half * colh

        pltpu.sync_copy(sizes_hbm, sz_v)
        sizes_vec = sz_v[...]
        starts_vec = lax.cumsum(sizes_vec, axis=0) - sizes_vec
        lane = lax.iota(jnp.int32, L)
        sel = lane == bag
        start = jnp.max(jnp.where(sel, starts_vec, 0))
        size = jnp.max(jnp.where(sel, sizes_vec, 0))

        n_full = size // CH
        rem = size - n_full * CH

        neg_inf = jnp.full((L,), -jnp.inf, dtype=jnp.float32)
        accs0 = (neg_inf,) * ngrp

        def chunk_body(i, accs):
            pltpu.async_copy(
                x_hbm.at[pl.ds(start + i * CH, CH), pl.ds(col0, colh)],
                buf.at[0], sem).wait()

            def row_body(r, accs):
                return tuple(
                    jnp.maximum(accs[j], buf[0, r, pl.ds(j * L, L)])
                    for j in range(ngrp))

            return lax.fori_loop(0, CH, row_body, accs)

        accs = lax.fori_loop(0, n_full, chunk_body, accs0)

        def rem_body(r, accs):
            pltpu.async_copy(
                x_hbm.at[pl.ds(start + n_full * CH + r, 1), pl.ds(col0, colh)],
                out_v, sem).wait()
            return tuple(
                jnp.maximum(accs[j], out_v[0, pl.ds(j * L, L)])
                for j in range(ngrp))

        accs = lax.fori_loop(0, rem, rem_body, accs)

        for j in range(ngrp):
            out_v[0, pl.ds(j * L, L)] = accs[j]
        pltpu.sync_copy(out_v, out_hbm.at[pl.ds(bag, 1), pl.ds(col0, colh)])

    return seg_max


def kernel(inter_pre, bags_size):
    total, d = inter_pre.shape
    b = bags_size.shape[0]
    assert b == L and d % (2 * L) == 0
    seg_max = _make_seg_max(total, d, b)
    return seg_max(inter_pre, bags_size.astype(jnp.int32))


# speedup vs baseline: 3.0667x; 1.1903x over previous
import functools

import jax
import jax.numpy as jnp
from jax import lax
from jax.experimental import pallas as pl
from jax.experimental.pallas import tpu as pltpu
from jax.experimental.pallas import tpu_sc as plsc

L = 16
CH = 128
RU = 4
NC = 2
NS = 16


def _make_seg_max(total, d, b):
    colh = d // 2
    ngrp = colh // L

    mesh = plsc.VectorSubcoreMesh(
        core_axis_name="c", subcore_axis_name="s",
        num_cores=NC, num_subcores=NS)

    @functools.partial(
        pl.kernel,
        out_type=jax.ShapeDtypeStruct((b, d), jnp.float32),
        mesh=mesh,
        scratch_types=[
            pltpu.VMEM((2, CH, colh), jnp.float32),
            pltpu.VMEM((L,), jnp.int32),
            pltpu.VMEM((1, colh), jnp.float32),
            pltpu.SemaphoreType.DMA,
            pltpu.SemaphoreType.DMA,
        ],
        compiler_params=pltpu.CompilerParams(
            use_tc_tiling_on_sc=False, needs_layout_passes=False),
    )
    def seg_max(x_hbm, sizes_hbm, out_hbm, buf, sz_v, out_v, sem0, sem1):
        wid = lax.axis_index("s") * NC + lax.axis_index("c")
        bag = wid // 2
        half = wid % 2
        col0 = half * colh

        pltpu.sync_copy(sizes_hbm, sz_v)
        sizes_vec = sz_v[...]
        starts_vec = lax.cumsum(sizes_vec, axis=0) - sizes_vec
        lane = lax.iota(jnp.int32, L)
        sel = lane == bag
        start = jnp.max(jnp.where(sel, starts_vec, 0))
        size = jnp.max(jnp.where(sel, sizes_vec, 0))

        n_full = size // CH
        rem = size - n_full * CH
        n_pair = n_full // 2

        def chunk_slice(i):
            return x_hbm.at[pl.ds(start + i * CH, CH), pl.ds(col0, colh)]

        sems = (sem0, sem1)

        def start_dma(i, k):
            pltpu.async_copy(chunk_slice(i), buf.at[k], sems[k])

        def wait_dma(i, k):
            pltpu.make_async_copy(chunk_slice(i), buf.at[k], sems[k]).wait()

        def reduce_chunk(k, accs):
            def body(r4, accs):
                r = r4 * RU
                for rr in range(RU):
                    accs = tuple(
                        jnp.maximum(accs[j], buf[k, r + rr, pl.ds(j * L, L)])
                        for j in range(ngrp))
                return accs
            return lax.fori_loop(0, CH // RU, body, accs)

        neg_inf = jnp.full((L,), -jnp.inf, dtype=jnp.float32)
        accs0 = (neg_inf,) * ngrp

        @pl.when(n_pair > 0)
        def _():
            start_dma(0, 0)

        def pair_body(p, accs):
            i0 = 2 * p
            start_dma(i0 + 1, 1)
            wait_dma(i0, 0)
            accs = reduce_chunk(0, accs)

            @pl.when(i0 + 2 < n_pair * 2)
            def _():
                start_dma(i0 + 2, 0)

            wait_dma(i0 + 1, 1)
            return reduce_chunk(1, accs)

        accs = lax.fori_loop(0, n_pair, pair_body, accs0)

        def odd_fn(accs):
            pltpu.async_copy(chunk_slice(n_pair * 2), buf.at[0], sem0).wait()
            return reduce_chunk(0, accs)

        accs = lax.cond(n_full % 2 == 1, odd_fn, lambda a: a, accs)

        def rem_body(r, accs):
            pltpu.async_copy(
                x_hbm.at[pl.ds(start + n_full * CH + r, 1), pl.ds(col0, colh)],
                out_v, sem0).wait()
            return tuple(
                jnp.maximum(accs[j], out_v[0, pl.ds(j * L, L)])
                for j in range(ngrp))

        accs = lax.fori_loop(0, rem, rem_body, accs)

        for j in range(ngrp):
            out_v[0, pl.ds(j * L, L)] = accs[j]
        pltpu.sync_copy(out_v, out_hbm.at[pl.ds(bag, 1), pl.ds(col0, colh)])

    return seg_max


def kernel(inter_pre, bags_size):
    total, d = inter_pre.shape
    b = bags_size.shape[0]
    assert b == L and d % (2 * L) == 0
    seg_max = _make_seg_max(total, d, b)
    return seg_max(inter_pre, bags_size.astype(jnp.int32))


# speedup vs baseline: 5.9016x; 1.9244x over previous
import functools

import jax
import jax.numpy as jnp
from jax import lax
from jax.experimental import pallas as pl
from jax.experimental.pallas import tpu as pltpu
from jax.experimental.pallas import tpu_sc as plsc

L = 16
CH = 128
RU = 4
NC = 2
NS = 16


def _make_seg_max(total, d, b):
    colh = d // 2
    ngrp = colh // L

    mesh = plsc.VectorSubcoreMesh(
        core_axis_name="c", subcore_axis_name="s",
        num_cores=NC, num_subcores=NS)

    @functools.partial(
        pl.kernel,
        out_type=jax.ShapeDtypeStruct((b * 8, d), jnp.float32),
        mesh=mesh,
        scratch_types=[
            pltpu.VMEM((2, CH, colh), jnp.float32),
            pltpu.VMEM((L,), jnp.int32),
            pltpu.VMEM((8, colh), jnp.float32),
            pltpu.SemaphoreType.DMA,
            pltpu.SemaphoreType.DMA,
        ],
        compiler_params=pltpu.CompilerParams(needs_layout_passes=False),
    )
    def seg_max(x_hbm, sizes_hbm, out_hbm, buf, sz_v, out_v, sem0, sem1):
        wid = lax.axis_index("s") * NC + lax.axis_index("c")
        bag = wid // 2
        half = wid % 2
        col0 = half * colh

        pltpu.sync_copy(sizes_hbm, sz_v)
        sizes_vec = sz_v[...]
        starts_vec = lax.cumsum(sizes_vec, axis=0) - sizes_vec
        lane = lax.iota(jnp.int32, L)
        sel = lane == bag
        start = pl.multiple_of(jnp.max(jnp.where(sel, starts_vec, 0)), 8)
        size = jnp.max(jnp.where(sel, sizes_vec, 0))

        n_full = size // CH
        rem = size - n_full * CH
        n_pair = n_full // 2

        def chunk_slice(i):
            return x_hbm.at[pl.ds(start + i * CH, CH), pl.ds(col0, colh)]

        sems = (sem0, sem1)

        def start_dma(i, k):
            pltpu.async_copy(chunk_slice(i), buf.at[k], sems[k])

        def wait_dma(i, k):
            pltpu.make_async_copy(chunk_slice(i), buf.at[k], sems[k]).wait()

        def reduce_chunk(k, accs):
            def body(r4, accs):
                r = r4 * RU
                for rr in range(RU):
                    accs = tuple(
                        jnp.maximum(accs[j], buf[k, r + rr, pl.ds(j * L, L)])
                        for j in range(ngrp))
                return accs
            return lax.fori_loop(0, CH // RU, body, accs)

        neg_inf = jnp.full((L,), -jnp.inf, dtype=jnp.float32)
        accs0 = (neg_inf,) * ngrp

        @pl.when(n_pair > 0)
        def _():
            start_dma(0, 0)

        def pair_body(p, accs):
            i0 = 2 * p
            start_dma(i0 + 1, 1)
            wait_dma(i0, 0)
            accs = reduce_chunk(0, accs)

            @pl.when(i0 + 2 < n_pair * 2)
            def _():
                start_dma(i0 + 2, 0)

            wait_dma(i0 + 1, 1)
            return reduce_chunk(1, accs)

        accs = lax.fori_loop(0, n_pair, pair_body, accs0)

        def odd_fn(accs):
            pltpu.async_copy(chunk_slice(n_pair * 2), buf.at[0], sem0).wait()
            return reduce_chunk(0, accs)

        accs = lax.cond(n_full % 2 == 1, odd_fn, lambda a: a, accs)

        def rem_body(r, accs):
            off = pl.multiple_of(start + n_full * CH + r, 8)
            pltpu.async_copy(
                x_hbm.at[pl.ds(off, 1), pl.ds(col0, colh)],
                out_v.at[pl.ds(0, 1)], sem0).wait()
            return tuple(
                jnp.maximum(accs[j], out_v[0, pl.ds(j * L, L)])
                for j in range(ngrp))

        accs = lax.fori_loop(0, rem, rem_body, accs)

        for j in range(ngrp):
            out_v[0, pl.ds(j * L, L)] = accs[j]
        row0 = pl.multiple_of(bag * 8, 8)
        pltpu.sync_copy(out_v, out_hbm.at[pl.ds(row0, 8), pl.ds(col0, colh)])

    return seg_max


def kernel(inter_pre, bags_size):
    total, d = inter_pre.shape
    b = bags_size.shape[0]
    assert b == L and d % (2 * L) == 0
    seg_max = _make_seg_max(total, d, b)
    out_pad = seg_max(inter_pre, bags_size.astype(jnp.int32))
    return out_pad[::8]


# speedup vs baseline: 6.0074x; 1.0179x over previous
import functools

import jax
import jax.numpy as jnp
from jax import lax
from jax.experimental import pallas as pl
from jax.experimental.pallas import tpu as pltpu
from jax.experimental.pallas import tpu_sc as plsc

L = 16
CH = 128
RU = 4
NC = 2
NS = 16


def _make_seg_max(total, d, b):
    colh = d // 2
    ngrp = colh // L

    mesh = plsc.VectorSubcoreMesh(
        core_axis_name="c", subcore_axis_name="s",
        num_cores=NC, num_subcores=NS)

    @functools.partial(
        pl.kernel,
        out_type=jax.ShapeDtypeStruct((b, d), jnp.float32),
        mesh=mesh,
        scratch_types=[
            pltpu.VMEM((2, CH, colh), jnp.float32),
            pltpu.VMEM((L,), jnp.int32),
            pltpu.VMEM((1, colh), jnp.float32),
            pltpu.VMEM_SHARED((NS, colh), jnp.float32),
            pltpu.SemaphoreType.DMA,
            pltpu.SemaphoreType.DMA,
        ],
        compiler_params=pltpu.CompilerParams(needs_layout_passes=False),
    )
    def seg_max(x_hbm, sizes_hbm, out_hbm, buf, sz_v, out_v, board, sem0, sem1):
        core = lax.axis_index("c")
        sub = lax.axis_index("s")
        bag = sub
        col0 = core * colh

        pltpu.sync_copy(sizes_hbm, sz_v)
        sizes_vec = sz_v[...]
        starts_vec = lax.cumsum(sizes_vec, axis=0) - sizes_vec
        lane = lax.iota(jnp.int32, L)
        sel = lane == bag
        start = jnp.max(jnp.where(sel, starts_vec, 0))
        size = jnp.max(jnp.where(sel, sizes_vec, 0))
        end = start + size

        a_lo = 8 * (start // 8)
        a_hi = 8 * ((end + 7) // 8)
        span = a_hi - a_lo
        n_ch = (span + CH - 1) // CH
        hi_base = jnp.maximum(a_hi - CH, 0)

        def chunk_base(i):
            return pl.multiple_of(jnp.minimum(a_lo + i * CH, hi_base), 8)

        def chunk_slice(i):
            return x_hbm.at[pl.ds(chunk_base(i), CH), pl.ds(col0, colh)]

        sems = (sem0, sem1)

        def start_dma(i, k):
            pltpu.async_copy(chunk_slice(i), buf.at[k], sems[k])

        def wait_dma(i, k):
            pltpu.make_async_copy(chunk_slice(i), buf.at[k], sems[k]).wait()

        neg_inf = jnp.full((L,), -jnp.inf, dtype=jnp.float32)

        def process_chunk(i, k, accs):
            base = chunk_base(i)
            head = jnp.clip(start - base, 0, CH)
            tail = jnp.clip(base + CH - end, 0, CH)

            def blank_head(r, _):
                for j in range(ngrp):
                    buf[k, r, pl.ds(j * L, L)] = neg_inf
                return 0

            def blank_tail(r, _):
                for j in range(ngrp):
                    buf[k, CH - 1 - r, pl.ds(j * L, L)] = neg_inf
                return 0

            lax.fori_loop(0, head, blank_head, 0)
            lax.fori_loop(0, tail, blank_tail, 0)

            def body(r4, accs):
                r = r4 * RU
                for rr in range(RU):
                    accs = tuple(
                        jnp.maximum(accs[j], buf[k, r + rr, pl.ds(j * L, L)])
                        for j in range(ngrp))
                return accs
            return lax.fori_loop(0, CH // RU, body, accs)

        accs0 = (neg_inf,) * ngrp
        n_pair = n_ch // 2

        @pl.when(n_pair > 0)
        def _():
            start_dma(0, 0)

        def pair_body(p, accs):
            i0 = 2 * p
            start_dma(i0 + 1, 1)
            wait_dma(i0, 0)
            accs = process_chunk(i0, 0, accs)

            @pl.when(i0 + 2 < n_pair * 2)
            def _():
                start_dma(i0 + 2, 0)

            wait_dma(i0 + 1, 1)
            return process_chunk(i0 + 1, 1, accs)

        accs = lax.fori_loop(0, n_pair, pair_body, accs0)

        def odd_fn(accs):
            pltpu.async_copy(chunk_slice(n_pair * 2), buf.at[0], sem0).wait()
            return process_chunk(n_pair * 2, 0, accs)

        accs = lax.cond(n_ch % 2 == 1, odd_fn, lambda a: a, accs)

        for j in range(ngrp):
            out_v[0, pl.ds(j * L, L)] = accs[j]
        pltpu.sync_copy(out_v, board.at[pl.ds(bag, 1)])
        plsc.subcore_barrier()

        @pl.when(sub < 4)
        def _():
            r0 = pl.multiple_of(8 * (sub // 2), 8)
            c0 = pl.multiple_of(col0 + 128 * (sub % 2), 128)
            pltpu.sync_copy(
                board.at[pl.ds(r0, 8), pl.ds(128 * (sub % 2), 128)],
                out_hbm.at[pl.ds(r0, 8), pl.ds(c0, 128)])

    return seg_max


def kernel(inter_pre, bags_size):
    total, d = inter_pre.shape
    b = bags_size.shape[0]
    assert b == L and d % (2 * L) == 0 and total % 8 == 0 and total >= CH
    seg_max = _make_seg_max(total, d, b)
    return seg_max(inter_pre, bags_size.astype(jnp.int32))


# speedup vs baseline: 6.0157x; 1.0014x over previous
import functools

import jax
import jax.numpy as jnp
from jax import lax
from jax.experimental import pallas as pl
from jax.experimental.pallas import tpu as pltpu
from jax.experimental.pallas import tpu_sc as plsc

L = 16
CH = 128
RU = 4
NC = 2
NS = 16


def _make_seg_max(total, d, b):
    colh = d // 2
    ngrp = colh // L

    mesh = plsc.VectorSubcoreMesh(
        core_axis_name="c", subcore_axis_name="s",
        num_cores=NC, num_subcores=NS)

    @functools.partial(
        pl.kernel,
        out_type=jax.ShapeDtypeStruct((b, d), jnp.float32),
        mesh=mesh,
        scratch_types=[
            pltpu.VMEM((2, CH, colh), jnp.float32),
            pltpu.VMEM((L,), jnp.int32),
            pltpu.VMEM((1, colh), jnp.float32),
            pltpu.VMEM_SHARED((NS, colh), jnp.float32),
            pltpu.SemaphoreType.DMA,
            pltpu.SemaphoreType.DMA,
        ],
        compiler_params=pltpu.CompilerParams(
            needs_layout_passes=False, skip_device_barrier=True),
    )
    def seg_max(x_hbm, sizes_hbm, out_hbm, buf, sz_v, out_v, board, sem0, sem1):
        core = lax.axis_index("c")
        sub = lax.axis_index("s")
        bag = sub
        col0 = core * colh

        pltpu.sync_copy(sizes_hbm, sz_v)
        sizes_vec = sz_v[...]
        starts_vec = lax.cumsum(sizes_vec, axis=0) - sizes_vec
        lane = lax.iota(jnp.int32, L)
        sel = lane == bag
        start = jnp.max(jnp.where(sel, starts_vec, 0))
        size = jnp.max(jnp.where(sel, sizes_vec, 0))
        end = start + size

        a_lo = 8 * (start // 8)
        a_hi = 8 * ((end + 7) // 8)
        span = a_hi - a_lo
        n_ch = (span + CH - 1) // CH
        hi_base = jnp.maximum(a_hi - CH, 0)

        def chunk_base(i):
            return pl.multiple_of(jnp.minimum(a_lo + i * CH, hi_base), 8)

        def chunk_slice(i):
            return x_hbm.at[pl.ds(chunk_base(i), CH), pl.ds(col0, colh)]

        sems = (sem0, sem1)

        def start_dma(i, k):
            pltpu.async_copy(chunk_slice(i), buf.at[k], sems[k])

        def wait_dma(i, k):
            pltpu.make_async_copy(chunk_slice(i), buf.at[k], sems[k]).wait()

        neg_inf = jnp.full((L,), -jnp.inf, dtype=jnp.float32)

        def process_chunk(i, k, accs):
            base = chunk_base(i)
            head = jnp.clip(start - base, 0, CH)
            tail = jnp.clip(base + CH - end, 0, CH)

            def blank_head(r, _):
                for j in range(ngrp):
                    buf[k, r, pl.ds(j * L, L)] = neg_inf
                return 0

            def blank_tail(r, _):
                for j in range(ngrp):
                    buf[k, CH - 1 - r, pl.ds(j * L, L)] = neg_inf
                return 0

            lax.fori_loop(0, head, blank_head, 0)
            lax.fori_loop(0, tail, blank_tail, 0)

            def body(r4, accs):
                r = r4 * RU
                for rr in range(RU):
                    accs = tuple(
                        jnp.maximum(accs[j], buf[k, r + rr, pl.ds(j * L, L)])
                        for j in range(ngrp))
                return accs
            return lax.fori_loop(0, CH // RU, body, accs)

        accs0 = (neg_inf,) * ngrp
        n_pair = n_ch // 2

        @pl.when(n_pair > 0)
        def _():
            start_dma(0, 0)

        def pair_body(p, accs):
            i0 = 2 * p
            start_dma(i0 + 1, 1)
            wait_dma(i0, 0)
            accs = process_chunk(i0, 0, accs)

            @pl.when(i0 + 2 < n_pair * 2)
            def _():
                start_dma(i0 + 2, 0)

            wait_dma(i0 + 1, 1)
            return process_chunk(i0 + 1, 1, accs)

        accs = lax.fori_loop(0, n_pair, pair_body, accs0)

        def odd_fn(accs):
            pltpu.async_copy(chunk_slice(n_pair * 2), buf.at[0], sem0).wait()
            return process_chunk(n_pair * 2, 0, accs)

        accs = lax.cond(n_ch % 2 == 1, odd_fn, lambda a: a, accs)

        for j in range(ngrp):
            out_v[0, pl.ds(j * L, L)] = accs[j]
        pltpu.sync_copy(out_v, board.at[pl.ds(bag, 1)])
        plsc.subcore_barrier()

        @pl.when(sub < 4)
        def _():
            r0 = pl.multiple_of(8 * (sub // 2), 8)
            c0 = pl.multiple_of(col0 + 128 * (sub % 2), 128)
            pltpu.sync_copy(
                board.at[pl.ds(r0, 8), pl.ds(128 * (sub % 2), 128)],
                out_hbm.at[pl.ds(r0, 8), pl.ds(c0, 128)])

    return seg_max


def kernel(inter_pre, bags_size):
    total, d = inter_pre.shape
    b = bags_size.shape[0]
    assert b == L and d % (2 * L) == 0 and total % 8 == 0 and total >= CH
    seg_max = _make_seg_max(total, d, b)
    return seg_max(inter_pre, bags_size.astype(jnp.int32))
